# baseline (speedup 1.0000x reference)
import functools

import jax
import jax.numpy as jnp
from jax import lax
from jax.experimental import pallas as pl
from jax.experimental.pallas import tpu as pltpu
from jax.experimental.pallas import tpu_sc as plsc

VOCAB = 1000000
D = 32
B = 4096
L = 200

NC, NS = 2, 16
NW = NC * NS
BPW = B // NW
NG = BPW // 16
FT, FS = D // 8, 8
CHL = 2
CHR = CHL * BPW
NCH = L // CHL
NBUF = 4
PER_W = L * BPW
PARTS = 4
PW = D // PARTS

_mesh = plsc.VectorSubcoreMesh(core_axis_name="c", subcore_axis_name="s")

_scratch = (
    [pltpu.VMEM((PER_W,), jnp.int32)]
    + [pltpu.VMEM((CHR, PW), jnp.float32)
       for _ in range(PARTS * NBUF)]
    + [pltpu.VMEM((CHR,), jnp.int32)
       for _ in range(PARTS * NBUF)]
    + [pltpu.VMEM((CHL, D, 16), jnp.float32)
       for _ in range(NBUF)]
    + [pltpu.VMEM((CHL, FT, FS, BPW), jnp.float32)
       for _ in range(NBUF)]
    + [pltpu.SemaphoreType.DMA for _ in range(2 * NBUF)]
)


@functools.partial(
    pl.kernel,
    mesh=_mesh,
    out_type=jax.ShapeDtypeStruct((L, FT, NW, FS, BPW), jnp.float32),
    compiler_params=pltpu.CompilerParams(
        use_tc_tiling_on_sc=False, needs_layout_passes=False),
    scratch_types=_scratch,
)
def _embed_sc(table_hbm, idx_hbm, peb_hbm, out_hbm, idx_v, *scr):
    n = PARTS * NBUF
    rows = [[scr[p * NBUF + b] for p in range(PARTS)] for b in range(NBUF)]
    gid = [[scr[n + p * NBUF + b] for p in range(PARTS)] for b in range(NBUF)]
    pb = scr[2 * n:2 * n + NBUF]
    ob = scr[2 * n + NBUF:2 * n + 2 * NBUF]
    sg = scr[2 * n + 2 * NBUF:2 * n + 3 * NBUF]
    so = scr[2 * n + 3 * NBUF:2 * n + 4 * NBUF]

    wid = lax.axis_index("s") * NC + lax.axis_index("c")

    pltpu.sync_copy(idx_hbm.at[pl.ds(wid * PER_W, PER_W)], idx_v)

    iota = lax.iota(jnp.int32, 16)
    pvec = [lax.broadcast(jnp.int32(p), (16,)) for p in range(PARTS)]

    def splat(v):
        return lax.broadcast(v, (16,))

    def start_gather(c, buf):
        for t in range(CHR // 16):
            v = idx_v[pl.ds(c * CHR + 16 * t, 16)]
            vs = v * PARTS
            for p in range(PARTS):
                gid[buf][p][pl.ds(16 * t, 16)] = vs + pvec[p]
        for p in range(PARTS):
            pltpu.async_copy(
                table_hbm.at[gid[buf][p]], rows[buf][p], sg[buf])
        pltpu.async_copy(peb_hbm.at[pl.ds(c * CHL, CHL)], pb[buf], so[buf])

    def wait_gather(c, buf):
        for p in range(PARTS):
            pltpu.make_async_copy(
                table_hbm.at[gid[buf][p]], rows[buf][p], sg[buf]).wait()
        pltpu.make_async_copy(
            peb_hbm.at[pl.ds(c * CHL, CHL)], pb[buf], so[buf]).wait()

    def start_write(c, buf):
        pltpu.async_copy(
            ob[buf], out_hbm.at[pl.ds(c * CHL, CHL), :, wid, :, :], so[buf])

    def wait_write(c, buf):
        pltpu.make_async_copy(
            ob[buf], out_hbm.at[pl.ds(c * CHL, CHL), :, wid, :, :], so[buf]
        ).wait()

    def transpose_emit(c, buf):
        for j in range(CHL):
            for p in range(PARTS):
                rp = rows[buf][p]

                def e_body(e2, carry, p=p, rp=rp, j=j):
                    for u in range(2):
                        e = e2 * 2 + u
                        f = p * PW + e
                        ps = pb[buf][j, f, pl.ds(0, 16)]
                        ft = f // FS
                        fs = f % FS
                        for g in range(NG):
                            kr = iota + (j * BPW + 16 * g)
                            cv = plsc.load_gather(rp, [kr, splat(e)])
                            ob[buf][j, ft, fs, pl.ds(16 * g, 16)] = cv + ps
                    return carry

                lax.fori_loop(0, PW // 2, e_body, 0)

    for b in range(NBUF):
        start_gather(b, b)

    def quad_body(q, carry):
        c0 = q * NBUF
        for b in range(NBUF):
            c = c0 + b
            wait_gather(c, b)
            transpose_emit(c, b)
            start_write(c, b)
            nxt = c + NBUF - 1
            prv = (b - 1) % NBUF

            @pl.when(jnp.logical_and(c >= 1, nxt < NCH))
            def _():
                wait_write(c - 1, prv)
                start_gather(nxt, prv)

        return carry

    lax.fori_loop(0, NCH // NBUF, quad_body, 0)

    for b in range(NBUF):
        c = NCH - NBUF + b
        wait_write(c, c % NBUF)


def kernel(x, table, pos_embedding):
    idx_w = (jnp.transpose(x).astype(jnp.int32)
             .reshape(L, NW, BPW).transpose(1, 0, 2).reshape(-1))
    t2 = table.reshape(VOCAB * PARTS, PW)
    pe_b = jnp.broadcast_to(
        pos_embedding[:L, :].astype(jnp.float32)[:, :, None], (L, D, 16))
    raw = _embed_sc(t2, idx_w, pe_b)
    return jnp.transpose(raw, (2, 4, 0, 1, 3)).reshape(B, L, D)


# speedup vs baseline: 1.1039x; 1.1039x over previous
import functools

import jax
import jax.numpy as jnp
from jax import lax
from jax.experimental import pallas as pl
from jax.experimental.pallas import tpu as pltpu
from jax.experimental.pallas import tpu_sc as plsc

VOCAB = 1000000
D = 32
B = 4096
L = 200

NC, NS = 2, 16
NW = NC * NS
BPW = B // NW
NG = BPW // 16
FT, FS = D // 8, 8
CHL = 2
CHR = CHL * BPW
NCH = L // CHL
NBUF = 5
PER_W = L * BPW
PARTS = 4
PW = D // PARTS

_mesh = plsc.VectorSubcoreMesh(core_axis_name="c", subcore_axis_name="s")

_scratch = (
    [pltpu.VMEM((PER_W,), jnp.int32)]
    + [pltpu.VMEM((CHR, PW), jnp.float32)
       for _ in range(PARTS * NBUF)]
    + [pltpu.VMEM((CHR,), jnp.int32)
       for _ in range(PARTS * NBUF)]
    + [pltpu.VMEM((CHL, D, 16), jnp.float32)
       for _ in range(NBUF)]
    + [pltpu.VMEM((CHL, FT, FS, BPW), jnp.float32)
       for _ in range(NBUF)]
    + [pltpu.SemaphoreType.DMA for _ in range(2 * NBUF)]
)


@functools.partial(
    pl.kernel,
    mesh=_mesh,
    out_type=jax.ShapeDtypeStruct((L, FT, NW, FS, BPW), jnp.float32),
    compiler_params=pltpu.CompilerParams(
        use_tc_tiling_on_sc=False, needs_layout_passes=False),
    scratch_types=_scratch,
)
def _embed_sc(table_hbm, idx_hbm, peb_hbm, out_hbm, idx_v, *scr):
    n = PARTS * NBUF
    rows = [[scr[p * NBUF + b] for p in range(PARTS)] for b in range(NBUF)]
    gid = [[scr[n + p * NBUF + b] for p in range(PARTS)] for b in range(NBUF)]
    pb = scr[2 * n:2 * n + NBUF]
    ob = scr[2 * n + NBUF:2 * n + 2 * NBUF]
    sg = scr[2 * n + 2 * NBUF:2 * n + 3 * NBUF]
    so = scr[2 * n + 3 * NBUF:2 * n + 4 * NBUF]

    wid = lax.axis_index("s") * NC + lax.axis_index("c")

    pltpu.sync_copy(idx_hbm.at[pl.ds(wid * PER_W, PER_W)], idx_v)

    iota = lax.iota(jnp.int32, 16)
    pvec = [lax.broadcast(jnp.int32(p), (16,)) for p in range(PARTS)]

    def splat(v):
        return lax.broadcast(v, (16,))

    def start_gather(c, buf):
        for t in range(CHR // 16):
            v = idx_v[pl.ds(c * CHR + 16 * t, 16)]
            vs = v * PARTS
            for p in range(PARTS):
                gid[buf][p][pl.ds(16 * t, 16)] = vs + pvec[p]
        for p in range(PARTS):
            pltpu.async_copy(
                table_hbm.at[gid[buf][p]], rows[buf][p], sg[buf])
        pltpu.async_copy(peb_hbm.at[pl.ds(c * CHL, CHL)], pb[buf], so[buf])

    def wait_gather(c, buf):
        for p in range(PARTS):
            pltpu.make_async_copy(
                table_hbm.at[gid[buf][p]], rows[buf][p], sg[buf]).wait()
        pltpu.make_async_copy(
            peb_hbm.at[pl.ds(c * CHL, CHL)], pb[buf], so[buf]).wait()

    def start_write(c, buf):
        pltpu.async_copy(
            ob[buf], out_hbm.at[pl.ds(c * CHL, CHL), :, wid, :, :], so[buf])

    def wait_write(c, buf):
        pltpu.make_async_copy(
            ob[buf], out_hbm.at[pl.ds(c * CHL, CHL), :, wid, :, :], so[buf]
        ).wait()

    def transpose_emit(c, buf):
        for j in range(CHL):
            for p in range(PARTS):
                rp = rows[buf][p]

                def e_body(e, carry, p=p, rp=rp, j=j):
                    f = p * PW + e
                    ps = pb[buf][j, f, pl.ds(0, 16)]
                    ft = f // FS
                    fs = f % FS
                    for g in range(NG):
                        kr = iota + (j * BPW + 16 * g)
                        cv = plsc.load_gather(rp, [kr, splat(e)])
                        ob[buf][j, ft, fs, pl.ds(16 * g, 16)] = cv + ps
                    return carry

                lax.fori_loop(0, PW, e_body, 0)

    for b in range(NBUF):
        start_gather(b, b)

    def quad_body(q, carry):
        c0 = q * NBUF
        for b in range(NBUF):
            c = c0 + b
            wait_gather(c, b)
            transpose_emit(c, b)
            start_write(c, b)
            nxt = c + NBUF - 1
            prv = (b - 1) % NBUF

            @pl.when(jnp.logical_and(c >= 1, nxt < NCH))
            def _():
                wait_write(c - 1, prv)
                start_gather(nxt, prv)

        return carry

    lax.fori_loop(0, NCH // NBUF, quad_body, 0)

    for b in range(NBUF):
        c = NCH - NBUF + b
        wait_write(c, c % NBUF)


def kernel(x, table, pos_embedding):
    idx_w = (jnp.transpose(x).astype(jnp.int32)
             .reshape(L, NW, BPW).transpose(1, 0, 2).reshape(-1))
    t2 = table.reshape(VOCAB * PARTS, PW)
    pe_b = jnp.broadcast_to(
        pos_embedding[:L, :].astype(jnp.float32)[:, :, None], (L, D, 16))
    raw = _embed_sc(t2, idx_w, pe_b)
    return jnp.transpose(raw, (2, 4, 0, 1, 3)).reshape(B, L, D)


# speedup vs baseline: 1.1058x; 1.0017x over previous
import functools

import jax
import jax.numpy as jnp
from jax import lax
from jax.experimental import pallas as pl
from jax.experimental.pallas import tpu as pltpu
from jax.experimental.pallas import tpu_sc as plsc

VOCAB = 1000000
D = 32
B = 4096
L = 200

NC, NS = 2, 16
NW = NC * NS
BPW = B // NW
NG = BPW // 16
FT, FS = D // 8, 8
CHL = 2
CHR = CHL * BPW
NCH = L // CHL
NBUF = 4
PER_W = L * BPW
PARTS = 4
PW = D // PARTS

_mesh = plsc.VectorSubcoreMesh(core_axis_name="c", subcore_axis_name="s")

_scratch = (
    [pltpu.VMEM((PER_W,), jnp.int32)]
    + [pltpu.VMEM((CHR, PW), jnp.float32)
       for _ in range(PARTS * NBUF)]
    + [pltpu.VMEM((CHR,), jnp.int32)
       for _ in range(PARTS * NBUF)]
    + [pltpu.VMEM((CHL, D, 16), jnp.float32)
       for _ in range(NBUF)]
    + [pltpu.VMEM((CHL, FT, FS, BPW), jnp.float32)
       for _ in range(NBUF)]
    + [pltpu.SemaphoreType.DMA for _ in range(2 * NBUF)]
)


@functools.partial(
    pl.kernel,
    mesh=_mesh,
    out_type=jax.ShapeDtypeStruct((L, FT, NW, FS, BPW), jnp.float32),
    compiler_params=pltpu.CompilerParams(
        use_tc_tiling_on_sc=False, needs_layout_passes=False),
    scratch_types=_scratch,
)
def _embed_sc(table_hbm, idx_hbm, peb_hbm, out_hbm, idx_v, *scr):
    n = PARTS * NBUF
    rows = [[scr[p * NBUF + b] for p in range(PARTS)] for b in range(NBUF)]
    gid = [[scr[n + p * NBUF + b] for p in range(PARTS)] for b in range(NBUF)]
    pb = scr[2 * n:2 * n + NBUF]
    ob = scr[2 * n + NBUF:2 * n + 2 * NBUF]
    sg = scr[2 * n + 2 * NBUF:2 * n + 3 * NBUF]
    so = scr[2 * n + 3 * NBUF:2 * n + 4 * NBUF]

    wid = lax.axis_index("s") * NC + lax.axis_index("c")

    pltpu.sync_copy(idx_hbm.at[pl.ds(wid * PER_W, PER_W)], idx_v)

    iota = lax.iota(jnp.int32, 16)
    pvec = [lax.broadcast(jnp.int32(p), (16,)) for p in range(PARTS)]

    def splat(v):
        return lax.broadcast(v, (16,))

    def start_gather(c, buf):
        for t in range(CHR // 16):
            v = idx_v[pl.ds(c * CHR + 16 * t, 16)]
            vs = v * PARTS
            for p in range(PARTS):
                gid[buf][p][pl.ds(16 * t, 16)] = vs + pvec[p]
        for p in range(PARTS):
            pltpu.async_copy(
                table_hbm.at[gid[buf][p]], rows[buf][p], sg[buf])
        pltpu.async_copy(peb_hbm.at[pl.ds(c * CHL, CHL)], pb[buf], so[buf])

    def wait_gather(c, buf):
        for p in range(PARTS):
            pltpu.make_async_copy(
                table_hbm.at[gid[buf][p]], rows[buf][p], sg[buf]).wait()
        pltpu.make_async_copy(
            peb_hbm.at[pl.ds(c * CHL, CHL)], pb[buf], so[buf]).wait()

    def start_write(c, buf):
        pltpu.async_copy(
            ob[buf], out_hbm.at[pl.ds(c * CHL, CHL), :, wid, :, :], so[buf])

    def wait_write(c, buf):
        pltpu.make_async_copy(
            ob[buf], out_hbm.at[pl.ds(c * CHL, CHL), :, wid, :, :], so[buf]
        ).wait()

    def transpose_emit(c, buf):
        for j in range(CHL):
            for p in range(PARTS):
                rp = rows[buf][p]

                def e_body(e, carry, p=p, rp=rp, j=j):
                    f = p * PW + e
                    ps = pb[buf][j, f, pl.ds(0, 16)]
                    ft = f // FS
                    fs = f % FS
                    for g in range(NG):
                        kr = iota + (j * BPW + 16 * g)
                        cv = plsc.load_gather(rp, [kr, splat(e)])
                        ob[buf][j, ft, fs, pl.ds(16 * g, 16)] = cv + ps
                    return carry

                lax.fori_loop(0, PW, e_body, 0)

    for b in range(NBUF):
        start_gather(b, b)

    def quad_body(q, carry):
        c0 = q * NBUF
        for b in range(NBUF):
            c = c0 + b
            wait_gather(c, b)
            transpose_emit(c, b)
            start_write(c, b)
            nxt = c + NBUF - 1
            prv = (b - 1) % NBUF

            @pl.when(jnp.logical_and(c >= 1, nxt < NCH))
            def _():
                wait_write(c - 1, prv)
                start_gather(nxt, prv)

        return carry

    lax.fori_loop(0, NCH // NBUF, quad_body, 0)

    for b in range(NBUF):
        c = NCH - NBUF + b
        wait_write(c, c % NBUF)


def kernel(x, table, pos_embedding):
    idx_w = (jnp.transpose(x).astype(jnp.int32)
             .reshape(L, NW, BPW).transpose(1, 0, 2).reshape(-1))
    t2 = table.reshape(VOCAB * PARTS, PW)
    pe_b = jnp.broadcast_to(
        pos_embedding[:L, :].astype(jnp.float32)[:, :, None], (L, D, 16))
    raw = _embed_sc(t2, idx_w, pe_b)
    return jnp.transpose(raw, (2, 4, 0, 1, 3)).reshape(B, L, D)
